# Initial kernel scaffold; baseline (speedup 1.0000x reference)
import jax
import jax.numpy as jnp
from jax.experimental import pallas as pl


def kernel(z, W):
    raise NotImplementedError("write your pallas kernel here")


# speedup vs baseline: 8.6105x; 8.6105x over previous
import functools

import jax
import jax.numpy as jnp
from jax import lax
from jax.experimental import pallas as pl
from jax.experimental.pallas import tpu as pltpu
from jax.experimental.pallas import tpu_sc as plsc

N = 16384
D = 256
K = 8192
BETA = 0.25

_NC = 2
_NS = 16
_NW = _NC * _NS
_BPW = N // _NW
_CH = 128
_NCH = _BPW // _CH


@functools.cache
def _make_sc_gather():
    mesh = plsc.VectorSubcoreMesh(core_axis_name="c", subcore_axis_name="s")

    @functools.partial(
        pl.kernel,
        out_type=jax.ShapeDtypeStruct((N, D), jnp.float32),
        mesh=mesh,
        scratch_types=[
            pltpu.VMEM((_BPW,), jnp.int32),
            pltpu.VMEM((_CH, D), jnp.float32),
            pltpu.VMEM((_CH, D), jnp.float32),
            pltpu.SemaphoreType.DMA,
            pltpu.SemaphoreType.DMA,
        ],
    )
    def sc_gather(w_hbm, idx_hbm, out_hbm, idx_v, buf0, buf1, sem0, sem1):
        wid = lax.axis_index("s") * _NC + lax.axis_index("c")
        base = wid * _BPW
        pltpu.sync_copy(idx_hbm.at[pl.ds(base, _BPW)], idx_v)
        bufs = (buf0, buf1)
        sems = (sem0, sem1)
        copies = [None] * _NCH

        def issue(c):
            copies[c] = pltpu.async_copy(
                w_hbm.at[idx_v.at[pl.ds(c * _CH, _CH)]], bufs[c % 2], sems[c % 2])

        issue(0)
        if _NCH > 1:
            issue(1)
        for c in range(_NCH):
            copies[c].wait()
            pltpu.sync_copy(bufs[c % 2], out_hbm.at[pl.ds(base + c * _CH, _CH)])
            if c + 2 < _NCH:
                issue(c + 2)

    return sc_gather


_LBN = 1024


def _loss_body(zq_ref, z_ref, out_ref):
    i = pl.program_id(0)
    diff = zq_ref[...] - z_ref[...]
    part = jnp.sum(diff * diff)

    @pl.when(i == 0)
    def _():
        out_ref[...] = jnp.zeros_like(out_ref)

    out_ref[...] = out_ref[...] + part


_loss_call = pl.pallas_call(
    _loss_body,
    grid=(N // _LBN,),
    in_specs=[
        pl.BlockSpec((_LBN, D), lambda i: (i, 0)),
        pl.BlockSpec((_LBN, D), lambda i: (i, 0)),
    ],
    out_specs=pl.BlockSpec((1, 1), lambda i: (0, 0)),
    out_shape=jax.ShapeDtypeStruct((1, 1), jnp.float32),
)


def kernel(z, W):
    z_flat = z.reshape(N, D)
    z2 = jnp.sum(z_flat ** 2, axis=1, keepdims=True)
    w2 = jnp.sum(W ** 2, axis=1)
    d = (z2 + w2) - 2.0 * jnp.matmul(z_flat, W.T)
    idx = jnp.argmin(d, axis=1)
    z_q = _make_sc_gather()(W, idx.astype(jnp.int32))
    lsum = _loss_call(z_q, z_flat)
    loss = lsum[0, 0] * ((1.0 + BETA) / (N * D))
    return z_q.reshape(z.shape), idx.reshape(N, 1), loss


# speedup vs baseline: 8.7324x; 1.0141x over previous
import functools

import jax
import jax.numpy as jnp
from jax import lax
from jax.experimental import pallas as pl
from jax.experimental.pallas import tpu as pltpu
from jax.experimental.pallas import tpu_sc as plsc

N = 16384
D = 256
K = 8192
BETA = 0.25

_NC = 2
_NS = 16
_NW = _NC * _NS
_BPW = N // _NW
_CH = 128
_NCH = _BPW // _CH


@functools.cache
def _make_sc_gather():
    mesh = plsc.VectorSubcoreMesh(core_axis_name="c", subcore_axis_name="s")

    @functools.partial(
        pl.kernel,
        out_type=jax.ShapeDtypeStruct((N, D), jnp.float32),
        mesh=mesh,
        scratch_types=[
            pltpu.VMEM((_BPW,), jnp.int32),
            pltpu.VMEM((_CH, D), jnp.float32),
            pltpu.VMEM((_CH, D), jnp.float32),
            pltpu.SemaphoreType.DMA,
            pltpu.SemaphoreType.DMA,
        ],
    )
    def sc_gather(w_hbm, idx_hbm, out_hbm, idx_v, buf0, buf1, sem0, sem1):
        wid = lax.axis_index("s") * _NC + lax.axis_index("c")
        base = wid * _BPW
        pltpu.sync_copy(idx_hbm.at[pl.ds(base, _BPW)], idx_v)
        bufs = (buf0, buf1)
        sems = (sem0, sem1)
        copies = [None] * _NCH

        def issue(c):
            copies[c] = pltpu.async_copy(
                w_hbm.at[idx_v.at[pl.ds(c * _CH, _CH)]], bufs[c % 2], sems[c % 2])

        issue(0)
        if _NCH > 1:
            issue(1)
        for c in range(_NCH):
            copies[c].wait()
            pltpu.sync_copy(bufs[c % 2], out_hbm.at[pl.ds(base + c * _CH, _CH)])
            if c + 2 < _NCH:
                issue(c + 2)

    return sc_gather


_LBN = 4096


def _loss_body(zq_ref, z_ref, out_ref):
    i = pl.program_id(0)
    diff = zq_ref[...] - z_ref[...]
    part = jnp.sum(diff * diff)

    @pl.when(i == 0)
    def _():
        out_ref[...] = jnp.zeros_like(out_ref)

    out_ref[...] = out_ref[...] + part


_loss_call = pl.pallas_call(
    _loss_body,
    grid=(N // _LBN,),
    in_specs=[
        pl.BlockSpec((_LBN, D), lambda i: (i, 0)),
        pl.BlockSpec((_LBN, D), lambda i: (i, 0)),
    ],
    out_specs=pl.BlockSpec((1, 1), lambda i: (0, 0)),
    out_shape=jax.ShapeDtypeStruct((1, 1), jnp.float32),
)


def kernel(z, W):
    z_flat = z.reshape(N, D)
    z2 = jnp.sum(z_flat ** 2, axis=1, keepdims=True)
    w2 = jnp.sum(W ** 2, axis=1)
    d = (z2 + w2) - 2.0 * jnp.matmul(z_flat, W.T)
    idx = jnp.argmin(d, axis=1)
    z_q = _make_sc_gather()(W, idx.astype(jnp.int32))
    lsum = _loss_call(z_q, z_flat)
    loss = lsum[0, 0] * ((1.0 + BETA) / (N * D))
    return z_q.reshape(z.shape), idx.reshape(N, 1), loss


# speedup vs baseline: 8.7340x; 1.0002x over previous
import functools

import jax
import jax.numpy as jnp
from jax import lax
from jax.experimental import pallas as pl
from jax.experimental.pallas import tpu as pltpu
from jax.experimental.pallas import tpu_sc as plsc

N = 16384
D = 256
K = 8192
BETA = 0.25

_NC = 2
_NS = 16
_NW = _NC * _NS
_BPW = N // _NW
_CH = 128
_NCH = _BPW // _CH


@functools.cache
def _make_sc_gather():
    mesh = plsc.VectorSubcoreMesh(core_axis_name="c", subcore_axis_name="s")

    @functools.partial(
        pl.kernel,
        out_type=jax.ShapeDtypeStruct((N, D), jnp.float32),
        mesh=mesh,
        scratch_types=[
            pltpu.VMEM((_BPW,), jnp.int32),
            pltpu.VMEM((_CH, D), jnp.float32),
            pltpu.VMEM((_CH, D), jnp.float32),
            pltpu.SemaphoreType.DMA,
            pltpu.SemaphoreType.DMA,
        ],
    )
    def sc_gather(w_hbm, idx_hbm, out_hbm, idx_v, buf0, buf1, sem0, sem1):
        wid = lax.axis_index("s") * _NC + lax.axis_index("c")
        base = wid * _BPW
        pltpu.sync_copy(idx_hbm.at[pl.ds(base, _BPW)], idx_v)
        bufs = (buf0, buf1)
        sems = (sem0, sem1)
        copies = [None] * _NCH

        def issue(c):
            copies[c] = pltpu.async_copy(
                w_hbm.at[idx_v.at[pl.ds(c * _CH, _CH)]], bufs[c % 2], sems[c % 2])

        issue(0)
        if _NCH > 1:
            issue(1)
        for c in range(_NCH):
            copies[c].wait()
            pltpu.sync_copy(bufs[c % 2], out_hbm.at[pl.ds(base + c * _CH, _CH)])
            if c + 2 < _NCH:
                issue(c + 2)

    return sc_gather


_LBN = 8192


def _loss_body(zq_ref, z_ref, out_ref):
    i = pl.program_id(0)
    diff = zq_ref[...] - z_ref[...]
    part = jnp.sum(diff * diff)

    @pl.when(i == 0)
    def _():
        out_ref[...] = jnp.zeros_like(out_ref)

    out_ref[...] = out_ref[...] + part


_loss_call = pl.pallas_call(
    _loss_body,
    grid=(N // _LBN,),
    in_specs=[
        pl.BlockSpec((_LBN, D), lambda i: (i, 0)),
        pl.BlockSpec((_LBN, D), lambda i: (i, 0)),
    ],
    out_specs=pl.BlockSpec((1, 1), lambda i: (0, 0)),
    out_shape=jax.ShapeDtypeStruct((1, 1), jnp.float32),
)


def kernel(z, W):
    z_flat = z.reshape(N, D)
    z2 = jnp.sum(z_flat ** 2, axis=1, keepdims=True)
    w2 = jnp.sum(W ** 2, axis=1)
    d = (z2 + w2) - 2.0 * jnp.matmul(z_flat, W.T)
    idx = jnp.argmin(d, axis=1)
    z_q = _make_sc_gather()(W, idx.astype(jnp.int32))
    lsum = _loss_call(z_q, z_flat)
    loss = lsum[0, 0] * ((1.0 + BETA) / (N * D))
    return z_q.reshape(z.shape), idx.reshape(N, 1), loss
